# Initial kernel scaffold; baseline (speedup 1.0000x reference)
import jax
import jax.numpy as jnp
from jax.experimental import pallas as pl


def kernel(weight, tensor0):
    raise NotImplementedError("write your pallas kernel here")


# speedup vs baseline: 1.4792x; 1.4792x over previous
import functools

import jax
import jax.numpy as jnp
from jax import lax
from jax.experimental import pallas as pl
from jax.experimental.pallas import tpu as pltpu
from jax.experimental.pallas import tpu_sc as plsc

VOCAB = 1000000
EMBED_DIM = 32
BATCH = 4096
HIST = 200

_info = plsc.get_sparse_core_info()
NC, NS = _info.num_cores, _info.num_subcores
NW = NC * NS
TOTAL = BATCH * HIST
PER_W = TOTAL // NW
CHUNK = 128
NCHUNK = PER_W // CHUNK
K = 8
NGROUP = NCHUNK // K

_mesh = plsc.VectorSubcoreMesh(core_axis_name="c", subcore_axis_name="s")


@functools.partial(
    pl.kernel,
    mesh=_mesh,
    out_type=jax.ShapeDtypeStruct((TOTAL, EMBED_DIM), jnp.float32),
    scratch_types=[
        pltpu.VMEM((NCHUNK, CHUNK), jnp.int32),
        pltpu.VMEM((K * CHUNK, EMBED_DIM), jnp.float32),
        pltpu.SemaphoreType.DMA,
    ],
    compiler_params=pltpu.CompilerParams(use_tc_tiling_on_sc=False),
)
def _embed(idx_hbm, table_hbm, out_hbm, idx_v, rows_v, sem):
    wid = lax.axis_index("s") * NC + lax.axis_index("c")
    base = wid * PER_W
    pltpu.sync_copy(idx_hbm.at[wid], idx_v)

    def group(g, carry):
        copies = [
            pltpu.async_copy(
                table_hbm.at[idx_v.at[g * K + j]],
                rows_v.at[pl.ds(j * CHUNK, CHUNK)],
                sem,
            )
            for j in range(K)
        ]
        for cp in copies:
            cp.wait()
        pltpu.sync_copy(rows_v, out_hbm.at[pl.ds(base + g * K * CHUNK, K * CHUNK)])
        return carry

    lax.fori_loop(0, NGROUP, group, 0)


def kernel(weight, tensor0):
    idx = tensor0.reshape(NW, NCHUNK, CHUNK)
    out = _embed(idx, weight)
    return out.reshape(BATCH, HIST, EMBED_DIM)


# speedup vs baseline: 1.4952x; 1.0108x over previous
import functools

import jax
import jax.numpy as jnp
from jax import lax
from jax.experimental import pallas as pl
from jax.experimental.pallas import tpu as pltpu
from jax.experimental.pallas import tpu_sc as plsc

VOCAB = 1000000
EMBED_DIM = 32
BATCH = 4096
HIST = 200

_info = plsc.get_sparse_core_info()
NC, NS = _info.num_cores, _info.num_subcores
NW = NC * NS
TOTAL = BATCH * HIST
PER_W = TOTAL // NW
CHUNK = 128
NCHUNK = PER_W // CHUNK
K = 10
GROUP = K * CHUNK
NGROUP = NCHUNK // K
NP = NGROUP // 2

_mesh = plsc.VectorSubcoreMesh(core_axis_name="c", subcore_axis_name="s")


@functools.partial(
    pl.kernel,
    mesh=_mesh,
    out_type=jax.ShapeDtypeStruct((TOTAL, EMBED_DIM), jnp.float32),
    scratch_types=[
        pltpu.VMEM((NCHUNK, CHUNK), jnp.int32),
        pltpu.VMEM((GROUP, EMBED_DIM), jnp.float32),
        pltpu.VMEM((GROUP, EMBED_DIM), jnp.float32),
        pltpu.SemaphoreType.DMA,
        pltpu.SemaphoreType.DMA,
        pltpu.SemaphoreType.DMA,
        pltpu.SemaphoreType.DMA,
    ],
    compiler_params=pltpu.CompilerParams(use_tc_tiling_on_sc=False),
)
def _embed(idx_hbm, table_hbm, out_hbm, idx_v, buf_a, buf_b, gsem_a, gsem_b,
           osem_a, osem_b):
    wid = lax.axis_index("s") * NC + lax.axis_index("c")
    base = wid * PER_W
    pltpu.sync_copy(idx_hbm.at[wid], idx_v)

    def fire_gathers(g, buf, sem):
        for j in range(K):
            pltpu.async_copy(
                table_hbm.at[idx_v.at[g * K + j]],
                buf.at[pl.ds(j * CHUNK, CHUNK)],
                sem,
            )

    def drain_gathers(buf, sem):
        pltpu.make_async_copy(table_hbm.at[pl.ds(0, GROUP)], buf, sem).wait()

    def fire_out(g, buf, sem):
        pltpu.async_copy(buf, out_hbm.at[pl.ds(base + g * GROUP, GROUP)], sem)

    def drain_out(buf, sem):
        pltpu.make_async_copy(buf, out_hbm.at[pl.ds(base, GROUP)], sem).wait()

    fire_gathers(0, buf_a, gsem_a)
    fire_gathers(1, buf_b, gsem_b)

    def step(p, carry):
        g0 = 2 * p
        drain_gathers(buf_a, gsem_a)
        fire_out(g0, buf_a, osem_a)
        drain_gathers(buf_b, gsem_b)
        fire_out(g0 + 1, buf_b, osem_b)

        @pl.when(p < NP - 1)
        def _refill():
            drain_out(buf_a, osem_a)
            fire_gathers(g0 + 2, buf_a, gsem_a)
            drain_out(buf_b, osem_b)
            fire_gathers(g0 + 3, buf_b, gsem_b)

        return carry

    lax.fori_loop(0, NP, step, 0)
    drain_out(buf_a, osem_a)
    drain_out(buf_b, osem_b)


def kernel(weight, tensor0):
    idx = tensor0.reshape(NW, NCHUNK, CHUNK)
    out = _embed(idx, weight)
    return out.reshape(BATCH, HIST, EMBED_DIM)
